# baseline (speedup 1.0000x reference)
import math

import jax
import jax.numpy as jnp
from jax.experimental import pallas as pl
from jax.experimental.pallas import tpu as pltpu

_K = 8
_N_REAL = 800
_N_ALL = 1000
_HW = 128 * 128
_LN2 = math.log(2.0)
_LOG2E = 1.0 / _LN2


def _bce_body(idx_ref, np_ref, p_hbm, m_hbm, s_ref, o_ref, pbuf, mbuf,
              psem, msem):
    npos = np_ref[0]

    def issue(k):
        s = jax.lax.rem(k, _K)
        src_i = idx_ref[k]
        pltpu.make_async_copy(p_hbm.at[src_i], pbuf.at[s], psem.at[s]).start()
        pltpu.make_async_copy(m_hbm.at[src_i], mbuf.at[s], msem.at[s]).start()

    def _pro(k, c):
        @pl.when(k < npos)
        def _():
            issue(k)
        return c

    jax.lax.fori_loop(0, _K, _pro, 0)

    def _step(k, acc):
        s = jax.lax.rem(k, _K)
        pltpu.make_async_copy(p_hbm.at[idx_ref[k]], pbuf.at[s],
                              psem.at[s]).wait()
        pltpu.make_async_copy(m_hbm.at[idx_ref[k]], mbuf.at[s],
                              msem.at[s]).wait()
        x = pbuf[s]
        m = mbuf[s]
        xs = x * _LOG2E
        t = jnp.log2(1.0 + jnp.exp2(xs))
        u = t - jnp.where(m >= 0.5, xs, 0.0)

        @pl.when(k + _K < npos)
        def _():
            issue(k + _K)

        return acc + u

    acc = jax.lax.fori_loop(0, npos, _step,
                            jnp.zeros((128, 128), jnp.float32))

    s = s_ref[...]
    posf = (s > 0.0).astype(jnp.float32)
    flat = (jax.lax.broadcasted_iota(jnp.int32, (8, 128), 0) * 128
            + jax.lax.broadcasted_iota(jnp.int32, (8, 128), 1))
    denom = jnp.sum(posf)
    pad_cnt = jnp.sum(jnp.where(flat >= _N_REAL, posf, 0.0))
    loss_sum = _LN2 * jnp.sum(acc)
    loss = (loss_sum + pad_cnt * (_HW * _LN2)) / denom
    o_ref[...] = jnp.reshape(loss, (1, 1))


def kernel(mask_preds, masks, scores):
    preds3 = mask_preds[0]
    masks3 = masks[0, :_N_REAL]
    scores_f = scores.reshape(-1)

    pos800 = scores_f[:_N_REAL] > 0.0
    idx = jnp.nonzero(pos800, size=_N_REAL, fill_value=0)[0].astype(jnp.int32)
    npos = jnp.sum(pos800).astype(jnp.int32).reshape(1)
    s_pad = jnp.pad(scores_f, (0, 1024 - _N_ALL),
                    constant_values=-1.0).reshape(8, 128)

    grid_spec = pltpu.PrefetchScalarGridSpec(
        num_scalar_prefetch=2,
        grid=(1,),
        in_specs=[
            pl.BlockSpec(memory_space=pl.ANY),
            pl.BlockSpec(memory_space=pl.ANY),
            pl.BlockSpec((8, 128), lambda i, *_: (0, 0)),
        ],
        out_specs=pl.BlockSpec((1, 1), lambda i, *_: (0, 0)),
        scratch_shapes=[
            pltpu.VMEM((_K, 128, 128), jnp.float32),
            pltpu.VMEM((_K, 128, 128), jnp.float32),
            pltpu.SemaphoreType.DMA((_K,)),
            pltpu.SemaphoreType.DMA((_K,)),
        ],
    )
    out = pl.pallas_call(
        _bce_body,
        grid_spec=grid_spec,
        out_shape=jax.ShapeDtypeStruct((1, 1), jnp.float32),
    )(idx, npos, preds3, masks3, s_pad)
    return out[0, 0]


# speedup vs baseline: 1.4361x; 1.4361x over previous
import math

import jax
import jax.numpy as jnp
from jax.experimental import pallas as pl
from jax.experimental.pallas import tpu as pltpu

_J = 16
_NB = 4
_N_REAL = 800
_N_ALL = 1000
_HW = 128 * 128
_LN2 = math.log(2.0)
_LOG2E = 1.0 / _LN2


def _bce_body(idx_ref, np_ref, p_hbm, m_hbm, s_ref, o_ref, pbuf, mbuf,
              psem, msem):
    npos = np_ref[0]
    nch = (npos + _J - 1) // _J

    def issue(c):
        b = jax.lax.rem(c, _NB)
        for j in range(_J):
            k = jnp.minimum(c * _J + j, jnp.maximum(npos - 1, 0))
            src_i = idx_ref[k]
            pltpu.make_async_copy(p_hbm.at[src_i], pbuf.at[b, j], psem).start()
            pltpu.make_async_copy(m_hbm.at[src_i], mbuf.at[b, j], msem).start()

    def _pro(c, carry):
        @pl.when(c < nch)
        def _():
            issue(c)
        return carry

    jax.lax.fori_loop(0, _NB, _pro, 0)

    def _step(c, acc):
        b = jax.lax.rem(c, _NB)
        pltpu.make_async_copy(p_hbm.at[pl.ds(0, _J)], pbuf.at[b], psem).wait()
        pltpu.make_async_copy(m_hbm.at[pl.ds(0, _J)], mbuf.at[b], msem).wait()
        for j in range(_J):
            x = pbuf[b, j]
            m = mbuf[b, j]
            wf = ((c * _J + j) < npos).astype(jnp.float32)
            xs = x * _LOG2E
            t = jnp.log2(1.0 + jnp.exp2(xs))
            u = t - jnp.where(m >= 0.5, xs, 0.0)
            acc = acc + wf * u

        @pl.when(c + _NB < nch)
        def _():
            issue(c + _NB)

        return acc

    acc = jax.lax.fori_loop(0, nch, _step,
                            jnp.zeros((128, 128), jnp.float32))

    s = s_ref[...]
    posf = (s > 0.0).astype(jnp.float32)
    flat = (jax.lax.broadcasted_iota(jnp.int32, (8, 128), 0) * 128
            + jax.lax.broadcasted_iota(jnp.int32, (8, 128), 1))
    denom = jnp.sum(posf)
    pad_cnt = jnp.sum(jnp.where(flat >= _N_REAL, posf, 0.0))
    loss_sum = _LN2 * jnp.sum(acc)
    loss = (loss_sum + pad_cnt * (_HW * _LN2)) / denom
    o_ref[...] = jnp.reshape(loss, (1, 1))


def kernel(mask_preds, masks, scores):
    preds3 = mask_preds[0]
    masks3 = masks[0, :_N_REAL]
    scores_f = scores.reshape(-1)

    pos800 = scores_f[:_N_REAL] > 0.0
    idx = jnp.nonzero(pos800, size=_N_REAL, fill_value=0)[0].astype(jnp.int32)
    npos = jnp.sum(pos800).astype(jnp.int32).reshape(1)
    s_pad = jnp.pad(scores_f, (0, 1024 - _N_ALL),
                    constant_values=-1.0).reshape(8, 128)

    grid_spec = pltpu.PrefetchScalarGridSpec(
        num_scalar_prefetch=2,
        grid=(1,),
        in_specs=[
            pl.BlockSpec(memory_space=pl.ANY),
            pl.BlockSpec(memory_space=pl.ANY),
            pl.BlockSpec((8, 128), lambda i, *_: (0, 0)),
        ],
        out_specs=pl.BlockSpec((1, 1), lambda i, *_: (0, 0)),
        scratch_shapes=[
            pltpu.VMEM((_NB, _J, 128, 128), jnp.float32),
            pltpu.VMEM((_NB, _J, 128, 128), jnp.float32),
            pltpu.SemaphoreType.DMA,
            pltpu.SemaphoreType.DMA,
        ],
    )
    out = pl.pallas_call(
        _bce_body,
        grid_spec=grid_spec,
        out_shape=jax.ShapeDtypeStruct((1, 1), jnp.float32),
    )(idx, npos, preds3, masks3, s_pad)
    return out[0, 0]


# speedup vs baseline: 1.5304x; 1.0656x over previous
import math

import jax
import jax.numpy as jnp
from jax.experimental import pallas as pl
from jax.experimental.pallas import tpu as pltpu

_J = 16
_NB = 4
_N_REAL = 800
_NCH = _N_REAL // _J
_N_ALL = 1000
_HW = 128 * 128
_LN2 = math.log(2.0)
_LOG2E = 1.0 / _LN2


def _bce_body(wts_ref, p_hbm, m_hbm, s_ref, o_ref, pbuf, mbuf, psem, msem):

    def issue(c):
        b = jax.lax.rem(c, _NB)
        pltpu.make_async_copy(p_hbm.at[pl.ds(c * _J, _J)], pbuf.at[b],
                              psem).start()
        pltpu.make_async_copy(m_hbm.at[pl.ds(c * _J, _J)], mbuf.at[b],
                              msem).start()

    for c in range(_NB):
        issue(c)

    def _step(c, acc):
        b = jax.lax.rem(c, _NB)
        pltpu.make_async_copy(p_hbm.at[pl.ds(0, _J)], pbuf.at[b], psem).wait()
        pltpu.make_async_copy(m_hbm.at[pl.ds(0, _J)], mbuf.at[b], msem).wait()
        for j in range(_J):
            x = pbuf[b, j]
            m = mbuf[b, j]
            wf = (wts_ref[c * _J + j] > 0).astype(jnp.float32)
            xs = x * _LOG2E
            t = jnp.log2(1.0 + jnp.exp2(xs))
            u = t - jnp.where(m >= 0.5, xs, 0.0)
            acc = acc + wf * u

        @pl.when(c + _NB < _NCH)
        def _():
            issue(c + _NB)

        return acc

    acc = jax.lax.fori_loop(0, _NCH, _step,
                            jnp.zeros((128, 128), jnp.float32), unroll=False)

    s = s_ref[...]
    posf = (s > 0.0).astype(jnp.float32)
    flat = (jax.lax.broadcasted_iota(jnp.int32, (8, 128), 0) * 128
            + jax.lax.broadcasted_iota(jnp.int32, (8, 128), 1))
    denom = jnp.sum(posf)
    pad_cnt = jnp.sum(jnp.where(flat >= _N_REAL, posf, 0.0))
    loss_sum = _LN2 * jnp.sum(acc)
    loss = (loss_sum + pad_cnt * (_HW * _LN2)) / denom
    o_ref[...] = jnp.reshape(loss, (1, 1))


def kernel(mask_preds, masks, scores):
    preds3 = mask_preds[0]
    masks3 = masks[0, :_N_REAL]
    scores_f = scores.reshape(-1)

    wts = (scores_f[:_N_REAL] > 0.0).astype(jnp.int32)
    s_pad = jnp.pad(scores_f, (0, 1024 - _N_ALL),
                    constant_values=-1.0).reshape(8, 128)

    grid_spec = pltpu.PrefetchScalarGridSpec(
        num_scalar_prefetch=1,
        grid=(1,),
        in_specs=[
            pl.BlockSpec(memory_space=pl.ANY),
            pl.BlockSpec(memory_space=pl.ANY),
            pl.BlockSpec((8, 128), lambda i, *_: (0, 0)),
        ],
        out_specs=pl.BlockSpec((1, 1), lambda i, *_: (0, 0)),
        scratch_shapes=[
            pltpu.VMEM((_NB, _J, 128, 128), jnp.float32),
            pltpu.VMEM((_NB, _J, 128, 128), jnp.float32),
            pltpu.SemaphoreType.DMA,
            pltpu.SemaphoreType.DMA,
        ],
    )
    out = pl.pallas_call(
        _bce_body,
        grid_spec=grid_spec,
        out_shape=jax.ShapeDtypeStruct((1, 1), jnp.float32),
    )(wts, preds3, masks3, s_pad)
    return out[0, 0]
